# baseline (speedup 1.0000x reference)
import functools

import jax
import jax.numpy as jnp
from jax import lax
from jax.experimental import pallas as pl
from jax.experimental.pallas import tpu as pltpu
from jax.experimental.pallas import tpu_sc as plsc

_VOCAB = 100000
_D = 50
_DP = 56
_LW = 128
_SL = 56
_BATCH = 16384
_B = _BATCH * _D

_info = plsc.get_sparse_core_info()
_NC, _NS = _info.num_cores, _info.num_subcores
_NW = _NC * _NS
_BPW = _B // _NW
_RPW = _BPW // _D
_NB = 4
_G = _NB * _D
_NG = _BPW // _G
_L = 16

_mesh = plsc.VectorSubcoreMesh(core_axis_name="c", subcore_axis_name="s")


@functools.partial(
    pl.kernel,
    out_type=jax.ShapeDtypeStruct((_BATCH, _SL, _LW), jnp.float32),
    mesh=_mesh,
    scratch_types=[
        pltpu.VMEM((_RPW, _D), jnp.int32),
        pltpu.VMEM((_NG, _G), jnp.int32),
        pltpu.VMEM((2, _G, _DP), jnp.float32),
        pltpu.VMEM((2, _G, _LW), jnp.float32),
        pltpu.SemaphoreType.DMA,
        pltpu.SemaphoreType.DMA,
        pltpu.SemaphoreType.DMA,
        pltpu.SemaphoreType.DMA,
    ],
    compiler_params=pltpu.CompilerParams(
        use_tc_tiling_on_sc=False, needs_layout_passes=False
    ),
)
def _gather(idx_hbm, table_hbm, out_hbm, idx_raw, idx_c, gbuf, wbuf,
            gsem0, gsem1, osem0, osem1):
    wid = lax.axis_index("s") * _NC + lax.axis_index("c")
    ebase = wid * _RPW
    gsems = (gsem0, gsem1)
    osems = (osem0, osem1)

    pltpu.sync_copy(idx_hbm.at[pl.ds(ebase, _RPW)], idx_raw)

    lane = lax.iota(jnp.int32, _L)

    def compact_idx(k, carry):
        e = k * _L + lane
        vals = plsc.load_gather(idx_raw, [e // _D, e % _D])
        idx_c[(k * _L) // _G, pl.ds((k * _L) % _G, _L)] = vals
        return carry

    lax.fori_loop(0, _BPW // _L, compact_idx, 0)

    def out_copies(buf, j, fn):
        for m in range(_NB):
            fn(
                wbuf.at[buf].at[pl.ds(m * _D, _D)],
                out_hbm.at[ebase + j * _NB + m, pl.ds(0, _D)],
                osems[buf],
            )

    pltpu.async_copy(table_hbm.at[idx_c.at[0]], gbuf.at[0], gsems[0])

    def outer(j2, carry):
        for b in range(2):
            j = j2 * 2 + b
            nb = 1 - b

            pltpu.make_async_copy(
                table_hbm.at[idx_c.at[j]], gbuf.at[b], gsems[b]
            ).wait()

            @pl.when(j + 1 < _NG)
            def _fire():
                pltpu.async_copy(
                    table_hbm.at[idx_c.at[j + 1]], gbuf.at[nb], gsems[nb]
                )

            @pl.when(j >= 2)
            def _drain():
                out_copies(
                    b, j - 2,
                    lambda s, d, sem: pltpu.make_async_copy(s, d, sem).wait(),
                )

            def repitch(q, carry2):
                wbuf[b, q, pl.ds(0, _L)] = gbuf[b, q, pl.ds(0, _L)]
                wbuf[b, q, pl.ds(_L, _L)] = gbuf[b, q, pl.ds(_L, _L)]
                wbuf[b, q, pl.ds(2 * _L, _L)] = gbuf[b, q, pl.ds(2 * _L, _L)]
                wbuf[b, q, pl.ds(_D - _L, _L)] = gbuf[b, q, pl.ds(_D - _L, _L)]
                return carry2

            lax.fori_loop(0, _G, repitch, 0)

            out_copies(b, j, pltpu.async_copy)
        return carry

    lax.fori_loop(0, _NG // 2, outer, 0)

    out_copies(0, _NG - 2,
               lambda s, d, sem: pltpu.make_async_copy(s, d, sem).wait())
    out_copies(1, _NG - 1,
               lambda s, d, sem: pltpu.make_async_copy(s, d, sem).wait())


def kernel(indices, table):
    table_p = jnp.pad(table, ((0, 0), (0, _DP - _D)))
    out = _gather(indices.astype(jnp.int32), table_p)
    return out[:, :_D, :_D]


# speedup vs baseline: 1.2374x; 1.2374x over previous
import functools

import jax
import jax.numpy as jnp
from jax import lax
from jax.experimental import pallas as pl
from jax.experimental.pallas import tpu as pltpu
from jax.experimental.pallas import tpu_sc as plsc

_VOCAB = 100000
_D = 50
_DP = 128
_SL = 56
_BATCH = 16384
_B = _BATCH * _D

_info = plsc.get_sparse_core_info()
_NC, _NS = _info.num_cores, _info.num_subcores
_NW = _NC * _NS
_BPW = _B // _NW
_RPW = _BPW // _D
_NB = 4
_G = _NB * _D
_NG = _BPW // _G
_L = 16

_mesh = plsc.VectorSubcoreMesh(core_axis_name="c", subcore_axis_name="s")


@functools.partial(
    pl.kernel,
    out_type=jax.ShapeDtypeStruct((_BATCH, _SL, _DP), jnp.float32),
    mesh=_mesh,
    scratch_types=[
        pltpu.VMEM((_RPW, _D), jnp.int32),
        pltpu.VMEM((_NG, _G), jnp.int32),
        pltpu.VMEM((2, _G, _DP), jnp.float32),
        pltpu.SemaphoreType.DMA,
        pltpu.SemaphoreType.DMA,
        pltpu.SemaphoreType.DMA,
        pltpu.SemaphoreType.DMA,
    ],
    compiler_params=pltpu.CompilerParams(
        use_tc_tiling_on_sc=False, needs_layout_passes=False
    ),
)
def _gather(idx_hbm, table_hbm, out_hbm, idx_raw, idx_c, gbuf,
            gsem0, gsem1, osem0, osem1):
    wid = lax.axis_index("s") * _NC + lax.axis_index("c")
    ebase = wid * _RPW
    gsems = (gsem0, gsem1)
    osems = (osem0, osem1)

    pltpu.sync_copy(idx_hbm.at[pl.ds(ebase, _RPW)], idx_raw)

    lane = lax.iota(jnp.int32, _L)

    def compact_idx(k, carry):
        e = k * _L + lane
        vals = plsc.load_gather(idx_raw, [e // _D, e % _D])
        idx_c[(k * _L) // _G, pl.ds((k * _L) % _G, _L)] = vals
        return carry

    lax.fori_loop(0, _BPW // _L, compact_idx, 0)

    def out_copies(buf, j, fn):
        for m in range(_NB):
            fn(
                gbuf.at[buf].at[pl.ds(m * _D, _D)],
                out_hbm.at[ebase + j * _NB + m, pl.ds(0, _D)],
                osems[buf],
            )

    pltpu.async_copy(table_hbm.at[idx_c.at[0]], gbuf.at[0], gsems[0])

    def outer(j2, carry):
        for b in range(2):
            j = j2 * 2 + b
            nb = 1 - b

            @pl.when(j + 1 < _NG)
            def _fire():
                @pl.when(j >= 1)
                def _drain():
                    out_copies(
                        nb, j - 1,
                        lambda s, d, sem: pltpu.make_async_copy(s, d, sem).wait(),
                    )

                pltpu.async_copy(
                    table_hbm.at[idx_c.at[j + 1]], gbuf.at[nb], gsems[nb]
                )

            pltpu.make_async_copy(
                table_hbm.at[idx_c.at[j]], gbuf.at[b], gsems[b]
            ).wait()
            out_copies(b, j, pltpu.async_copy)
        return carry

    lax.fori_loop(0, _NG // 2, outer, 0)

    out_copies(0, _NG - 2,
               lambda s, d, sem: pltpu.make_async_copy(s, d, sem).wait())
    out_copies(1, _NG - 1,
               lambda s, d, sem: pltpu.make_async_copy(s, d, sem).wait())


def kernel(indices, table):
    table_p = jnp.pad(table, ((0, 0), (0, _DP - _D)))
    out = _gather(indices.astype(jnp.int32), table_p)
    return out[:, :_D, :_D]
